# baseline (speedup 1.0000x reference)
import jax
import jax.numpy as jnp
from jax import lax
from jax.experimental import pallas as pl
from jax.experimental.pallas import tpu as pltpu
from jax.experimental.pallas import tpu_sc as plsc

BATCH = 16384
EMBED = 16
NUM_CORES = 2
NUM_SUBCORES = 16
NUM_WORKERS = NUM_CORES * NUM_SUBCORES
BPW = BATCH // NUM_WORKERS
CHUNK = 128
NCHUNK = BPW // CHUNK
NGROUP = BPW // EMBED


def _fm_body(user_hbm, item_hbm, user_mf_hbm, item_mf_hbm, u_bias_hbm,
             i_bias_hbm, wb_hbm, bc_hbm, out_hbm,
             idx_u, idx_i, u_rows, i_rows, ub, ib, out_v, wb_v, bc_v, sem):
    wid = lax.axis_index("s") * NUM_CORES + lax.axis_index("c")

    pltpu.sync_copy(user_hbm.at[wid], idx_u)
    pltpu.sync_copy(item_hbm.at[wid], idx_i)
    pltpu.sync_copy(wb_hbm, wb_v)
    pltpu.sync_copy(bc_hbm, bc_v)

    copies = []
    for k in range(NCHUNK):
        sl = pl.ds(k * CHUNK, CHUNK)
        copies.append(pltpu.async_copy(
            user_mf_hbm.at[idx_u.at[k]], u_rows.at[sl], sem))
        copies.append(pltpu.async_copy(
            item_mf_hbm.at[idx_i.at[k]], i_rows.at[sl], sem))
        copies.append(pltpu.async_copy(
            u_bias_hbm.at[idx_u.at[k]], ub.at[sl], sem))
        copies.append(pltpu.async_copy(
            i_bias_hbm.at[idx_i.at[k]], ib.at[sl], sem))
    for c in copies:
        c.wait()

    bc = bc_v[...]
    ws = [wb_v[e, :] for e in range(EMBED)]
    lanes = lax.iota(jnp.int32, EMBED)

    def group(g, carry):
        base16 = g * EMBED
        rows = base16 + lanes
        acc = bc + ub[pl.ds(base16, EMBED)] + ib[pl.ds(base16, EMBED)]
        for e in range(EMBED):
            ev = jnp.full((EMBED,), e, jnp.int32)
            gu = plsc.load_gather(u_rows, [rows, ev])
            gi = plsc.load_gather(i_rows, [rows, ev])
            acc = acc + gu * gi * ws[e]
        out_v[pl.ds(base16, EMBED)] = acc
        return carry

    lax.fori_loop(0, NGROUP, group, 0)

    pltpu.sync_copy(out_v, out_hbm.at[pl.ds(wid * BPW, BPW)])


@jax.jit
def _fm(user3d, item3d, user_mf, item_mf, u_bias1d, i_bias1d, wbcast, bc):
    mesh = plsc.VectorSubcoreMesh(core_axis_name="c", subcore_axis_name="s")
    return pl.kernel(
        _fm_body,
        out_type=jax.ShapeDtypeStruct((BATCH,), jnp.float32),
        mesh=mesh,
        compiler_params=pltpu.CompilerParams(
            needs_layout_passes=False, use_tc_tiling_on_sc=False),
        scratch_types=[
            pltpu.VMEM((NCHUNK, CHUNK), jnp.int32),
            pltpu.VMEM((NCHUNK, CHUNK), jnp.int32),
            pltpu.VMEM((BPW, EMBED), jnp.float32),
            pltpu.VMEM((BPW, EMBED), jnp.float32),
            pltpu.VMEM((BPW,), jnp.float32),
            pltpu.VMEM((BPW,), jnp.float32),
            pltpu.VMEM((BPW,), jnp.float32),
            pltpu.VMEM((EMBED, EMBED), jnp.float32),
            pltpu.VMEM((EMBED,), jnp.float32),
            pltpu.SemaphoreType.DMA,
        ],
    )(user3d, item3d, user_mf, item_mf, u_bias1d, i_bias1d, wbcast, bc)


def kernel(user, item, user_mf, item_mf, u_bias, i_bias, W, b, gb):
    user3d = user.reshape(NUM_WORKERS, NCHUNK, CHUNK)
    item3d = item.reshape(NUM_WORKERS, NCHUNK, CHUNK)
    wbcast = jnp.broadcast_to(W.reshape(EMBED, 1), (EMBED, EMBED))
    bc = jnp.full((EMBED,), b[0] + gb, dtype=jnp.float32)
    out = _fm(user3d, item3d, user_mf, item_mf,
              u_bias.reshape(-1), i_bias.reshape(-1), wbcast, bc)
    return out.reshape(BATCH, 1)
